# baseline (speedup 1.0000x reference)
import dataclasses

import jax
import jax.numpy as jnp
from jax import lax
from jax.experimental import pallas as pl
from jax.experimental.pallas import tpu as pltpu
from jax.experimental.pallas import tpu_sc as plsc

N = 10000
D = 128
OUT = 128
E = 320000

NC = 2
NS = 16
NW = NC * NS
K = 128
G = E // K
GC = G // NC
ZK = 80
ZCH = N // ZK
MAXI = GC // NS + 1


def _sc_body(h_hbm, ei_hbm, agg_hbm, cnt_hbm,
             idx0, idx1, idx2, idx3, rows0, rows1, cnt_v, agg_sh,
             gsem0, gsem1, isem0, isem1, isem2, isem3, ssem0, ssem1):
    _ZERO16 = jnp.zeros((16,), jnp.float32)
    _ONE16 = jnp.ones((16,), jnp.float32)
    c = lax.axis_index("c")
    s = lax.axis_index("s")
    wid = c * NS + s

    def _early(r, idx_c, isem):
        @pl.when(r < GC)
        def _():
            pltpu.async_copy(ei_hbm.at[c * GC + r], idx_c, isem)

    _early(s, idx0, isem0)
    _early(s + NS, idx1, isem1)
    _early(s + 2 * NS, idx2, isem2)
    _early(s + 3 * NS, idx3, isem3)

    @pl.when(s < GC)
    def _():
        pltpu.make_async_copy(ei_hbm.at[c * GC + s], idx0, isem0).wait()
        pltpu.async_copy(h_hbm.at[idx0.at[0]], rows0, gsem0)

    @pl.loop(0, K)
    def _(r):
        @pl.loop(0, D, step=16)
        def _(q):
            rows1[r, pl.ds(q, 16)] = _ZERO16

    @pl.loop(0, N, step=16)
    def _(q):
        cnt_v[pl.ds(q, 16)] = _ZERO16

    @pl.loop(0, 8)
    def _(i):
        ch = s + i * NS

        @pl.when(ch < ZCH)
        def _():
            pltpu.sync_copy(rows1.at[pl.ds(0, ZK)],
                            agg_sh.at[pl.ds(ch * ZK, ZK)])

    @pl.when(s + NS < GC)
    def _():
        pltpu.make_async_copy(ei_hbm.at[c * GC + s + NS], idx1, isem1).wait()
        pltpu.async_copy(h_hbm.at[idx1.at[0]], rows1, gsem1)

    plsc.subcore_barrier()

    def prefetch_idx(r, idx_c, isem):
        @pl.when(r < GC)
        def _():
            pltpu.async_copy(ei_hbm.at[c * GC + r], idx_c, isem)

    def start_gather(r, idx_c, isem, rows_v, gsem):
        @pl.when(r < GC)
        def _():
            pltpu.make_async_copy(ei_hbm.at[c * GC + r], idx_c, isem).wait()
            pltpu.async_copy(h_hbm.at[idx_c.at[0]], rows_v, gsem)

    def proc_chunk(r, idx_c, rows_v, gsem, ssem):
        @pl.when(r < GC)
        def _():
            @pl.loop(0, K, step=16)
            def _(q):
                plsc.addupdate_scatter(cnt_v, [idx_c[1, pl.ds(q, 16)]],
                                       _ONE16)

            pltpu.make_async_copy(h_hbm.at[idx_c.at[0]], rows_v, gsem).wait()
            pltpu.async_copy(rows_v, agg_sh.at[idx_c.at[1]], ssem, add=True)

    def wait_scatter(r, idx_c, rows_v, ssem):
        @pl.when(r < GC)
        def _():
            pltpu.make_async_copy(rows_v, agg_sh.at[idx_c.at[1]], ssem).wait()

    @pl.loop(0, MAXI + 1, step=4)
    def _(i):
        r = s + i * NS
        proc_chunk(r, idx0, rows0, gsem0, ssem0)
        proc_chunk(r + NS, idx1, rows1, gsem1, ssem1)
        wait_scatter(r, idx0, rows0, ssem0)
        prefetch_idx(r + 4 * NS, idx0, isem0)
        start_gather(r + 2 * NS, idx2, isem2, rows0, gsem0)
        wait_scatter(r + NS, idx1, rows1, ssem1)
        prefetch_idx(r + 5 * NS, idx1, isem1)
        start_gather(r + 3 * NS, idx3, isem3, rows1, gsem1)
        proc_chunk(r + 2 * NS, idx2, rows0, gsem0, ssem0)
        proc_chunk(r + 3 * NS, idx3, rows1, gsem1, ssem1)
        wait_scatter(r + 2 * NS, idx2, rows0, ssem0)
        prefetch_idx(r + 6 * NS, idx2, isem2)
        start_gather(r + 4 * NS, idx0, isem0, rows0, gsem0)
        wait_scatter(r + 3 * NS, idx3, rows1, ssem1)
        prefetch_idx(r + 7 * NS, idx3, isem3)
        start_gather(r + 5 * NS, idx1, isem1, rows1, gsem1)

    plsc.subcore_barrier()

    pltpu.sync_copy(cnt_v, cnt_hbm.at[wid])

    @pl.loop(0, 8)
    def _(i):
        ch = s + i * NS

        @pl.when(ch < ZCH)
        def _():
            pltpu.sync_copy(agg_sh.at[pl.ds(ch * ZK, ZK)],
                            agg_hbm.at[c, pl.ds(ch * ZK, ZK)])


def _sc_aggregate(h2, ei3):
    mesh = plsc.VectorSubcoreMesh(core_axis_name="c", subcore_axis_name="s")
    cp = pltpu.CompilerParams()
    if "needs_layout_passes" in pltpu.CompilerParams.__dataclass_fields__:
        cp = dataclasses.replace(cp, needs_layout_passes=False)
    fn = pl.kernel(
        _sc_body,
        compiler_params=cp,
        out_type=[
            jax.ShapeDtypeStruct((NC, N, D), jnp.float32),
            jax.ShapeDtypeStruct((NW, N), jnp.float32),
        ],
        mesh=mesh,
        scratch_types=[
            pltpu.VMEM((8, K), jnp.int32),
            pltpu.VMEM((8, K), jnp.int32),
            pltpu.VMEM((8, K), jnp.int32),
            pltpu.VMEM((8, K), jnp.int32),
            pltpu.VMEM((K, D), jnp.float32),
            pltpu.VMEM((K, D), jnp.float32),
            pltpu.VMEM((N,), jnp.float32),
            pltpu.VMEM_SHARED((N, D), jnp.float32),
            pltpu.SemaphoreType.DMA,
            pltpu.SemaphoreType.DMA,
            pltpu.SemaphoreType.DMA,
            pltpu.SemaphoreType.DMA,
            pltpu.SemaphoreType.DMA,
            pltpu.SemaphoreType.DMA,
            pltpu.SemaphoreType.DMA,
            pltpu.SemaphoreType.DMA,
        ],
    )
    return fn(h2, ei3)


def _tc_self_body(h_ref, ws_ref, b_ref, o_ref):
    dn = (((1,), (1,)), ((), ()))
    o_ref[...] = lax.dot_general(h_ref[...], ws_ref[...], dn,
                                 preferred_element_type=jnp.float32,
                                 precision=lax.Precision.HIGHEST) + b_ref[...]


def _tc_self(h2, W_self, bias2):
    R = 2000
    return pl.pallas_call(
        _tc_self_body,
        grid=(N // R,),
        in_specs=[
            pl.BlockSpec((R, D), lambda i: (i, 0)),
            pl.BlockSpec((OUT, D), lambda i: (0, 0)),
            pl.BlockSpec((1, OUT), lambda i: (0, 0)),
        ],
        out_specs=pl.BlockSpec((R, OUT), lambda i: (i, 0)),
        out_shape=jax.ShapeDtypeStruct((N, OUT), jnp.float32),
    )(h2, W_self, bias2)


def _tc_body(hs_ref, agg_ref, cnt_ref, wm_ref, o_ref):
    agg = agg_ref[0] + agg_ref[1]
    cnt = jnp.maximum(jnp.sum(cnt_ref[...], axis=1), 1.0)[:, None]
    dn = (((1,), (1,)), ((), ()))
    ha = lax.dot_general(agg, wm_ref[...], dn,
                         preferred_element_type=jnp.float32,
                         precision=lax.Precision.HIGHEST)
    o_ref[...] = jnp.maximum(hs_ref[...] + ha / cnt, 0.0)


def _tc_combine(hself, aggp, cntp, W_msg):
    R = 2000
    return pl.pallas_call(
        _tc_body,
        grid=(N // R,),
        in_specs=[
            pl.BlockSpec((R, OUT), lambda i: (i, 0)),
            pl.BlockSpec((NC, R, D), lambda i: (0, i, 0)),
            pl.BlockSpec((R, NW), lambda i: (i, 0)),
            pl.BlockSpec((OUT, D), lambda i: (0, 0)),
        ],
        out_specs=pl.BlockSpec((R, OUT), lambda i: (i, 0)),
        out_shape=jax.ShapeDtypeStruct((N, OUT), jnp.float32),
    )(hself, aggp, cntp, W_msg)


def kernel(h, edge_index, W_self, W_msg, bias):
    h2 = h.reshape(N, D)
    ei_t = jnp.transpose(edge_index.reshape(2, G, K), (1, 0, 2))
    ei3 = jnp.zeros((G, 8, K), jnp.int32).at[:, :2, :].set(ei_t)
    aggp, cntp = _sc_aggregate(h2, ei3)
    hself = _tc_self(h2, W_self, bias.reshape(1, OUT))
    out = _tc_combine(hself, aggp, cntp.T, W_msg)
    return out.reshape(1, N, OUT)


# speedup vs baseline: 1.2450x; 1.2450x over previous
import dataclasses

import jax
import jax.numpy as jnp
from jax import lax
from jax.experimental import pallas as pl
from jax.experimental.pallas import tpu as pltpu
from jax.experimental.pallas import tpu_sc as plsc

N = 10000
D = 128
OUT = 128
E = 320000

NC = 2
NS = 16
NW = NC * NS
K = 128
G = E // K
GC = G // NC
ZK = 80
ZCH = N // ZK
MAXI = GC // NS + 1


def _sc_body(h_hbm, ei_hbm, agg_hbm, cnt_hbm,
             idx0, idx1, idx2, idx3, rows0, rows1, cnt_v, agg_sh,
             gsem0, gsem1, isem0, isem1, isem2, isem3):
    _ZERO16 = jnp.zeros((16,), jnp.float32)
    _ONE16 = jnp.ones((16,), jnp.float32)
    c = lax.axis_index("c")
    s = lax.axis_index("s")
    wid = c * NS + s

    def _early(r, idx_c, isem):
        @pl.when(r < GC)
        def _():
            pltpu.async_copy(ei_hbm.at[c * GC + r], idx_c, isem)

    _early(s, idx0, isem0)
    _early(s + NS, idx1, isem1)

    @pl.when(s < GC)
    def _():
        pltpu.make_async_copy(ei_hbm.at[c * GC + s], idx0, isem0).wait()
        pltpu.async_copy(h_hbm.at[idx0.at[0]], rows0, gsem0)

    @pl.loop(0, K)
    def _(r):
        @pl.loop(0, D, step=16)
        def _(q):
            rows1[r, pl.ds(q, 16)] = _ZERO16

    @pl.loop(0, N, step=16)
    def _(q):
        cnt_v[pl.ds(q, 16)] = _ZERO16

    @pl.loop(0, 8)
    def _(i):
        ch = s + i * NS

        @pl.when(ch < ZCH)
        def _():
            pltpu.sync_copy(rows1.at[pl.ds(0, ZK)],
                            agg_sh.at[pl.ds(ch * ZK, ZK)])

    plsc.subcore_barrier()

    def prefetch_idx(r, idx_c, isem):
        @pl.when(r < GC)
        def _():
            pltpu.async_copy(ei_hbm.at[c * GC + r], idx_c, isem)

    def start_gather(r, idx_c, isem, rows_v, gsem):
        @pl.when(r < GC)
        def _():
            pltpu.make_async_copy(ei_hbm.at[c * GC + r], idx_c, isem).wait()
            pltpu.async_copy(h_hbm.at[idx_c.at[0]], rows_v, gsem)

    def finish_chunk(r, idx_c, rows_v, gsem):
        @pl.when(r < GC)
        def _():
            @pl.loop(0, K, step=16)
            def _(q):
                plsc.addupdate_scatter(cnt_v, [idx_c[1, pl.ds(q, 16)]],
                                       _ONE16)

            pltpu.make_async_copy(h_hbm.at[idx_c.at[0]], rows_v, gsem).wait()
            pltpu.sync_copy(rows_v, agg_sh.at[idx_c.at[1]], add=True)

    @pl.loop(0, MAXI + 1, step=4)
    def _(i):
        r = s + i * NS
        prefetch_idx(r + 2 * NS, idx2, isem2)
        prefetch_idx(r + 3 * NS, idx3, isem3)
        start_gather(r + NS, idx1, isem1, rows1, gsem1)
        finish_chunk(r, idx0, rows0, gsem0)
        start_gather(r + 2 * NS, idx2, isem2, rows0, gsem0)
        finish_chunk(r + NS, idx1, rows1, gsem1)
        prefetch_idx(r + 4 * NS, idx0, isem0)
        prefetch_idx(r + 5 * NS, idx1, isem1)
        start_gather(r + 3 * NS, idx3, isem3, rows1, gsem1)
        finish_chunk(r + 2 * NS, idx2, rows0, gsem0)
        start_gather(r + 4 * NS, idx0, isem0, rows0, gsem0)
        finish_chunk(r + 3 * NS, idx3, rows1, gsem1)

    plsc.subcore_barrier()

    pltpu.sync_copy(cnt_v, cnt_hbm.at[wid])

    @pl.loop(0, 8)
    def _(i):
        ch = s + i * NS

        @pl.when(ch < ZCH)
        def _():
            pltpu.sync_copy(agg_sh.at[pl.ds(ch * ZK, ZK)],
                            agg_hbm.at[c, pl.ds(ch * ZK, ZK)])


def _sc_aggregate(h2, ei3):
    mesh = plsc.VectorSubcoreMesh(core_axis_name="c", subcore_axis_name="s")
    cp = pltpu.CompilerParams()
    if "needs_layout_passes" in pltpu.CompilerParams.__dataclass_fields__:
        cp = dataclasses.replace(cp, needs_layout_passes=False)
    fn = pl.kernel(
        _sc_body,
        compiler_params=cp,
        out_type=[
            jax.ShapeDtypeStruct((NC, N, D), jnp.float32),
            jax.ShapeDtypeStruct((NW, N), jnp.float32),
        ],
        mesh=mesh,
        scratch_types=[
            pltpu.VMEM((8, K), jnp.int32),
            pltpu.VMEM((8, K), jnp.int32),
            pltpu.VMEM((8, K), jnp.int32),
            pltpu.VMEM((8, K), jnp.int32),
            pltpu.VMEM((K, D), jnp.float32),
            pltpu.VMEM((K, D), jnp.float32),
            pltpu.VMEM((N,), jnp.float32),
            pltpu.VMEM_SHARED((N, D), jnp.float32),
            pltpu.SemaphoreType.DMA,
            pltpu.SemaphoreType.DMA,
            pltpu.SemaphoreType.DMA,
            pltpu.SemaphoreType.DMA,
            pltpu.SemaphoreType.DMA,
            pltpu.SemaphoreType.DMA,
        ],
    )
    return fn(h2, ei3)


def _tc_self_body(h_ref, ws_ref, b_ref, o_ref):
    dn = (((1,), (1,)), ((), ()))
    o_ref[...] = lax.dot_general(h_ref[...], ws_ref[...], dn,
                                 preferred_element_type=jnp.float32,
                                 precision=lax.Precision.HIGHEST) + b_ref[...]


def _tc_self(h2, W_self, bias2):
    R = 2000
    return pl.pallas_call(
        _tc_self_body,
        grid=(N // R,),
        in_specs=[
            pl.BlockSpec((R, D), lambda i: (i, 0)),
            pl.BlockSpec((OUT, D), lambda i: (0, 0)),
            pl.BlockSpec((1, OUT), lambda i: (0, 0)),
        ],
        out_specs=pl.BlockSpec((R, OUT), lambda i: (i, 0)),
        out_shape=jax.ShapeDtypeStruct((N, OUT), jnp.float32),
    )(h2, W_self, bias2)


def _tc_body(hs_ref, agg_ref, cnt_ref, wm_ref, o_ref):
    agg = agg_ref[0] + agg_ref[1]
    cnt = jnp.maximum(jnp.sum(cnt_ref[...], axis=1), 1.0)[:, None]
    dn = (((1,), (1,)), ((), ()))
    ha = lax.dot_general(agg, wm_ref[...], dn,
                         preferred_element_type=jnp.float32,
                         precision=lax.Precision.HIGHEST)
    o_ref[...] = jnp.maximum(hs_ref[...] + ha / cnt, 0.0)


def _tc_combine(hself, aggp, cntp, W_msg):
    R = 2000
    return pl.pallas_call(
        _tc_body,
        grid=(N // R,),
        in_specs=[
            pl.BlockSpec((R, OUT), lambda i: (i, 0)),
            pl.BlockSpec((NC, R, D), lambda i: (0, i, 0)),
            pl.BlockSpec((R, NW), lambda i: (i, 0)),
            pl.BlockSpec((OUT, D), lambda i: (0, 0)),
        ],
        out_specs=pl.BlockSpec((R, OUT), lambda i: (i, 0)),
        out_shape=jax.ShapeDtypeStruct((N, OUT), jnp.float32),
    )(hself, aggp, cntp, W_msg)


def kernel(h, edge_index, W_self, W_msg, bias):
    h2 = h.reshape(N, D)
    ei_t = jnp.transpose(edge_index.reshape(2, G, K), (1, 0, 2))
    ei3 = jnp.zeros((G, 8, K), jnp.int32).at[:, :2, :].set(ei_t)
    aggp, cntp = _sc_aggregate(h2, ei3)
    hself = _tc_self(h2, W_self, bias.reshape(1, OUT))
    out = _tc_combine(hself, aggp, cntp.T, W_msg)
    return out.reshape(1, N, OUT)
